# baseline (speedup 1.0000x reference)
import functools

import jax
import jax.numpy as jnp
from jax import lax
from jax.experimental import pallas as pl
from jax.experimental.pallas import tpu as pltpu
from jax.experimental.pallas import tpu_sc as plsc

N_ROWS = 65536
N_COLS = 512
LANES = 16
NC = 2
NS = 16
NW = NC * NS
ROWS_PER_W = N_ROWS // NW
IN_R = 64
OUT_R = 32
N_CHUNKS = ROWS_PER_W // IN_R
BLKS = N_COLS // LANES


@functools.partial(
    pl.kernel,
    out_type=jax.ShapeDtypeStruct((N_ROWS, N_COLS), jnp.float32),
    mesh=plsc.VectorSubcoreMesh(core_axis_name="c", subcore_axis_name="s"),
    compiler_params=pltpu.CompilerParams(
        needs_layout_passes=False,
        disable_bounds_checks=True,
        disable_semaphore_checks=True,
    ),
    scratch_types=(
        [pltpu.VMEM((N_COLS,), jnp.int32)]
        + [pltpu.VMEM((IN_R, N_COLS), jnp.float32)] * 2
        + [pltpu.VMEM((OUT_R, N_COLS), jnp.float32)] * 2
        + [pltpu.SemaphoreType.DMA] * 4
    ),
)
def _permute_sc(x_hbm, perm_hbm, out_hbm, perm_v,
                in_v0, in_v1, out_v0, out_v1,
                sem_i0, sem_i1, sem_o0, sem_o1):
    in_bufs = (in_v0, in_v1)
    out_bufs = (out_v0, out_v1)
    in_sems = (sem_i0, sem_i1)
    out_sems = (sem_o0, sem_o1)

    wid = lax.axis_index("s") * NC + lax.axis_index("c")
    pltpu.sync_copy(perm_hbm, perm_v)
    row0 = wid * ROWS_PER_W

    def start_in(chunk, slot):
        pltpu.make_async_copy(
            x_hbm.at[pl.ds(row0 + chunk * IN_R, IN_R), :],
            in_bufs[slot], in_sems[slot]).start()

    start_in(0, 0)

    def permute_half(in_v, out_v, h):
        idxs = [perm_v[pl.ds(LANES * k, LANES)] for k in range(BLKS)]

        @plsc.parallel_loop(0, OUT_R, unroll=1)
        def _(r):
            row_idx = jnp.full((LANES,), h * OUT_R + r, dtype=jnp.int32)
            for k in range(BLKS):
                out_v[r, pl.ds(LANES * k, LANES)] = (
                    plsc.load_gather(in_v, [row_idx, idxs[k]]))

    def chunk_body(i, carry):
        for pi in range(2):
            chunk = 2 * i + pi
            in_v, sem_i = in_bufs[pi], in_sems[pi]

            pltpu.make_async_copy(
                x_hbm.at[pl.ds(0, IN_R), :], in_v, sem_i).wait()

            @pl.when(chunk + 1 < N_CHUNKS)
            def _():
                start_in(chunk + 1, (pi + 1) % 2)

            for h in range(2):
                out_v, sem_o = out_bufs[h], out_sems[h]

                @pl.when(chunk >= 1)
                def _():
                    pltpu.make_async_copy(
                        out_v, out_hbm.at[pl.ds(0, OUT_R), :], sem_o).wait()

                permute_half(in_v, out_v, h)
                pltpu.make_async_copy(
                    out_v,
                    out_hbm.at[
                        pl.ds(row0 + chunk * IN_R + h * OUT_R, OUT_R), :],
                    sem_o).start()
        return carry

    lax.fori_loop(0, N_CHUNKS // 2, chunk_body, 0)

    for h in range(2):
        pltpu.make_async_copy(
            out_bufs[h], out_hbm.at[pl.ds(0, OUT_R), :], out_sems[h]).wait()


def kernel(x, perm):
    return _permute_sc(x, perm)
